# Initial kernel scaffold; baseline (speedup 1.0000x reference)
import jax
import jax.numpy as jnp
from jax.experimental import pallas as pl


def kernel(x, fc_weight, bias):
    raise NotImplementedError("write your pallas kernel here")


# speedup vs baseline: 1.3036x; 1.3036x over previous
import functools

import jax
import jax.numpy as jnp
from jax import lax
from jax.experimental import pallas as pl
from jax.experimental.pallas import tpu as pltpu
from jax.experimental.pallas import tpu_sc as plsc

BATCH = 16384
NUM_FIELDS = 26
NUM_WORKERS = 32
BPW = BATCH // NUM_WORKERS
IDX_PER_W = BPW * NUM_FIELDS
CHUNK = 128
NCHUNK = IDX_PER_W // CHUNK

_mesh = plsc.VectorSubcoreMesh(core_axis_name="c", subcore_axis_name="s")


@functools.partial(
    pl.kernel,
    mesh=_mesh,
    out_type=jax.ShapeDtypeStruct((BATCH,), jnp.float32),
    compiler_params=pltpu.CompilerParams(needs_layout_passes=False),
    scratch_types=[
        pltpu.VMEM((NCHUNK, CHUNK), jnp.int32),
        pltpu.VMEM((IDX_PER_W,), jnp.float32),
        pltpu.VMEM((BPW,), jnp.float32),
        pltpu.SemaphoreType.DMA,
    ],
)
def _emb_sum(x_hbm, tbl_hbm, out_hbm, idx_v, vals_v, out_v, sem):
    wid = lax.axis_index("s") * 2 + lax.axis_index("c")

    pltpu.sync_copy(x_hbm.at[pl.ds(wid * NCHUNK, NCHUNK)], idx_v)

    def fire(j, c):
        dst = vals_v.at[pl.ds(pl.multiple_of(j * CHUNK, CHUNK), CHUNK)]
        pltpu.async_copy(tbl_hbm.at[idx_v.at[j]], dst, sem)
        return c

    lax.fori_loop(0, NCHUNK, fire, 0)

    def drain(j, c):
        dst = vals_v.at[pl.ds(pl.multiple_of(j * CHUNK, CHUNK), CHUNK)]
        pltpu.make_async_copy(tbl_hbm.at[idx_v.at[j]], dst, sem).wait()
        return c

    lax.fori_loop(0, NCHUNK, drain, 0)

    lanes = lax.broadcasted_iota(jnp.int32, (16,), 0)

    def red(i, c):
        base_lin = (i * 16 + lanes) * NUM_FIELDS
        acc = jnp.zeros((16,), jnp.float32)
        for f in range(NUM_FIELDS):
            acc = acc + plsc.load_gather(vals_v, [base_lin + f])
        out_v[pl.ds(i * 16, 16)] = acc
        return c

    lax.fori_loop(0, BPW // 16, red, 0)

    pltpu.sync_copy(out_v, out_hbm.at[pl.ds(wid * BPW, BPW)])


def kernel(x, fc_weight, bias):
    xw = x.astype(jnp.int32).reshape(NUM_WORKERS * NCHUNK, CHUNK)
    table = fc_weight.reshape(-1)
    out = _emb_sum(xw, table)
    return out.reshape(BATCH, 1) + bias


# speedup vs baseline: 1.3561x; 1.0403x over previous
import functools

import jax
import jax.numpy as jnp
from jax import lax
from jax.experimental import pallas as pl
from jax.experimental.pallas import tpu as pltpu
from jax.experimental.pallas import tpu_sc as plsc

BATCH = 16384
NUM_FIELDS = 26
NUM_WORKERS = 32
BPW = BATCH // NUM_WORKERS
IDX_PER_W = BPW * NUM_FIELDS
CHUNK = 128
NCHUNK = IDX_PER_W // CHUNK

TABLE = 1000000
STAGE_A = 62528
STAGE_B = TABLE - 15 * STAGE_A
SUB_A = STAGE_A // 4
SUB_B = STAGE_B // 4

_mesh = plsc.VectorSubcoreMesh(core_axis_name="c", subcore_axis_name="s")


@functools.partial(
    pl.kernel,
    mesh=_mesh,
    out_type=jax.ShapeDtypeStruct((BATCH,), jnp.float32),
    compiler_params=pltpu.CompilerParams(needs_layout_passes=False),
    scratch_types=[
        pltpu.VMEM((NCHUNK, CHUNK), jnp.int32),
        pltpu.VMEM((IDX_PER_W,), jnp.float32),
        pltpu.VMEM((BPW,), jnp.float32),
        pltpu.VMEM((SUB_A,), jnp.float32),
        pltpu.VMEM_SHARED((TABLE,), jnp.float32),
        pltpu.SemaphoreType.DMA,
    ],
)
def _emb_sum(x_hbm, tbl_hbm, out_hbm, idx_v, vals_v, out_v, bounce_v, tbl_sh,
             sem):
    sid = lax.axis_index("s")
    wid = sid * 2 + lax.axis_index("c")

    pltpu.sync_copy(x_hbm.at[pl.ds(wid * NCHUNK, NCHUNK)], idx_v)

    @pl.when(sid < 15)
    def _():
        for k in range(4):
            off = pl.multiple_of(sid * STAGE_A + k * SUB_A, 8)
            pltpu.sync_copy(tbl_hbm.at[pl.ds(off, SUB_A)], bounce_v)
            pltpu.sync_copy(bounce_v, tbl_sh.at[pl.ds(off, SUB_A)])

    @pl.when(sid == 15)
    def _():
        for k in range(4):
            off = 15 * STAGE_A + k * SUB_B
            pltpu.sync_copy(tbl_hbm.at[pl.ds(off, SUB_B)],
                            bounce_v.at[pl.ds(0, SUB_B)])
            pltpu.sync_copy(bounce_v.at[pl.ds(0, SUB_B)],
                            tbl_sh.at[pl.ds(off, SUB_B)])

    plsc.subcore_barrier()

    def fire(j, c):
        dst = vals_v.at[pl.ds(pl.multiple_of(j * CHUNK, CHUNK), CHUNK)]
        pltpu.async_copy(tbl_sh.at[idx_v.at[j]], dst, sem)
        return c

    lax.fori_loop(0, NCHUNK, fire, 0)

    def drain(j, c):
        dst = vals_v.at[pl.ds(pl.multiple_of(j * CHUNK, CHUNK), CHUNK)]
        pltpu.make_async_copy(tbl_sh.at[idx_v.at[j]], dst, sem).wait()
        return c

    lax.fori_loop(0, NCHUNK, drain, 0)

    lanes = lax.broadcasted_iota(jnp.int32, (16,), 0)

    def red(i, c):
        base_lin = (i * 16 + lanes) * NUM_FIELDS
        acc = jnp.zeros((16,), jnp.float32)
        for f in range(NUM_FIELDS):
            acc = acc + plsc.load_gather(vals_v, [base_lin + f])
        out_v[pl.ds(i * 16, 16)] = acc
        return c

    lax.fori_loop(0, BPW // 16, red, 0)

    pltpu.sync_copy(out_v, out_hbm.at[pl.ds(wid * BPW, BPW)])


def kernel(x, fc_weight, bias):
    xw = x.astype(jnp.int32).reshape(NUM_WORKERS * NCHUNK, CHUNK)
    table = fc_weight.reshape(-1)
    out = _emb_sum(xw, table)
    return out.reshape(BATCH, 1) + bias


# speedup vs baseline: 1.3575x; 1.0010x over previous
import functools

import jax
import jax.numpy as jnp
from jax import lax
from jax.experimental import pallas as pl
from jax.experimental.pallas import tpu as pltpu
from jax.experimental.pallas import tpu_sc as plsc

BATCH = 16384
NUM_FIELDS = 26
NUM_WORKERS = 32
BPW = BATCH // NUM_WORKERS
IDX_PER_W = BPW * NUM_FIELDS
CHUNK = 128
NCHUNK = IDX_PER_W // CHUNK

TABLE = 1000000
STAGE_A = 62528
STAGE_B = TABLE - 15 * STAGE_A
SUB_A = STAGE_A // 4
SUB_B = STAGE_B // 4

_mesh = plsc.VectorSubcoreMesh(core_axis_name="c", subcore_axis_name="s")


@functools.partial(
    pl.kernel,
    mesh=_mesh,
    out_type=jax.ShapeDtypeStruct((BATCH,), jnp.float32),
    compiler_params=pltpu.CompilerParams(needs_layout_passes=False),
    scratch_types=[
        pltpu.VMEM((IDX_PER_W,), jnp.int32),
        pltpu.VMEM((IDX_PER_W,), jnp.float32),
        pltpu.VMEM((BPW,), jnp.float32),
        pltpu.VMEM((SUB_A,), jnp.float32),
        pltpu.VMEM_SHARED((TABLE,), jnp.float32),
        pltpu.SemaphoreType.DMA,
    ],
)
def _emb_sum(x_hbm, tbl_hbm, out_hbm, idx_v, vals_v, out_v, bounce_v, tbl_sh,
             sem):
    sid = lax.axis_index("s")
    wid = sid * 2 + lax.axis_index("c")

    pltpu.sync_copy(x_hbm.at[pl.ds(pl.multiple_of(wid * IDX_PER_W, 8),
                                   IDX_PER_W)], idx_v)

    @pl.when(sid < 15)
    def _():
        for k in range(4):
            off = pl.multiple_of(sid * STAGE_A + k * SUB_A, 8)
            pltpu.sync_copy(tbl_hbm.at[pl.ds(off, SUB_A)], bounce_v)
            pltpu.sync_copy(bounce_v, tbl_sh.at[pl.ds(off, SUB_A)])

    @pl.when(sid == 15)
    def _():
        for k in range(4):
            off = 15 * STAGE_A + k * SUB_B
            pltpu.sync_copy(tbl_hbm.at[pl.ds(off, SUB_B)],
                            bounce_v.at[pl.ds(0, SUB_B)])
            pltpu.sync_copy(bounce_v.at[pl.ds(0, SUB_B)],
                            tbl_sh.at[pl.ds(off, SUB_B)])

    plsc.subcore_barrier()

    pltpu.async_copy(tbl_sh.at[idx_v], vals_v, sem).wait()

    lanes = lax.broadcasted_iota(jnp.int32, (16,), 0)

    def red(i, c):
        base_lin = (i * 16 + lanes) * NUM_FIELDS
        acc = jnp.zeros((16,), jnp.float32)
        for f in range(NUM_FIELDS):
            acc = acc + plsc.load_gather(vals_v, [base_lin + f])
        out_v[pl.ds(i * 16, 16)] = acc
        return c

    lax.fori_loop(0, BPW // 16, red, 0)

    pltpu.sync_copy(out_v, out_hbm.at[pl.ds(wid * BPW, BPW)])


def kernel(x, fc_weight, bias):
    xw = x.astype(jnp.int32).reshape(-1)
    table = fc_weight.reshape(-1)
    out = _emb_sum(xw, table)
    return out.reshape(BATCH, 1) + bias


# speedup vs baseline: 2.6031x; 1.9176x over previous
import functools

import jax
import jax.numpy as jnp
from jax import lax
from jax.experimental import pallas as pl
from jax.experimental.pallas import tpu as pltpu
from jax.experimental.pallas import tpu_sc as plsc

BATCH = 16384
NUM_FIELDS = 26
NUM_WORKERS = 32
BPW = BATCH // NUM_WORKERS
IDX_PER_W = BPW * NUM_FIELDS

TABLE = 1000000
STAGE = 62464
SUB = STAGE // 4
TAIL = TABLE - 15 * STAGE - 3 * SUB

_mesh = plsc.VectorSubcoreMesh(core_axis_name="c", subcore_axis_name="s")


@functools.partial(
    pl.kernel,
    mesh=_mesh,
    out_type=jax.ShapeDtypeStruct((BATCH,), jnp.float32),
    compiler_params=pltpu.CompilerParams(needs_layout_passes=False),
    scratch_types=[
        pltpu.VMEM((IDX_PER_W,), jnp.int32),
        pltpu.VMEM((IDX_PER_W,), jnp.float32),
        pltpu.VMEM((BPW,), jnp.float32),
        pltpu.VMEM((TAIL,), jnp.float32),
        pltpu.VMEM_SHARED((TABLE,), jnp.float32),
        pltpu.SemaphoreType.DMA,
    ],
)
def _emb_sum(xt_hbm, tbl_hbm, out_hbm, idx_v, vals_v, out_v, bounce_v, tbl_sh,
             sem):
    sid = lax.axis_index("s")
    wid = sid * 2 + lax.axis_index("c")
    base = wid * BPW

    for f in range(NUM_FIELDS):
        pltpu.sync_copy(xt_hbm.at[f, pl.ds(pl.multiple_of(base, 128), BPW)],
                        idx_v.at[pl.ds(f * BPW, BPW)])

    @pl.when(sid < 15)
    def _():
        for k in range(4):
            off = pl.multiple_of(sid * STAGE + k * SUB, 128)
            pltpu.sync_copy(tbl_hbm.at[0, pl.ds(off, SUB)],
                            bounce_v.at[pl.ds(0, SUB)])
            pltpu.sync_copy(bounce_v.at[pl.ds(0, SUB)],
                            tbl_sh.at[pl.ds(off, SUB)])

    @pl.when(sid == 15)
    def _():
        for k in range(3):
            off = 15 * STAGE + k * SUB
            pltpu.sync_copy(tbl_hbm.at[0, pl.ds(off, SUB)],
                            bounce_v.at[pl.ds(0, SUB)])
            pltpu.sync_copy(bounce_v.at[pl.ds(0, SUB)],
                            tbl_sh.at[pl.ds(off, SUB)])
        off = 15 * STAGE + 3 * SUB
        pltpu.sync_copy(tbl_hbm.at[0, pl.ds(off, TAIL)], bounce_v)
        pltpu.sync_copy(bounce_v, tbl_sh.at[pl.ds(off, TAIL)])

    plsc.subcore_barrier()

    pltpu.async_copy(tbl_sh.at[idx_v], vals_v, sem).wait()

    def red(i, c):
        b0 = i * 16
        acc = jnp.zeros((16,), jnp.float32)
        for f in range(NUM_FIELDS):
            acc = acc + vals_v[pl.ds(f * BPW + b0, 16)]
        out_v[pl.ds(b0, 16)] = acc
        return c

    lax.fori_loop(0, BPW // 16, red, 0)

    pltpu.sync_copy(out_v, out_hbm.at[pl.ds(base, BPW)])


def kernel(x, fc_weight, bias):
    xt = x.astype(jnp.int32).T
    tbl = fc_weight.T
    out = _emb_sum(xt, tbl)
    return out.reshape(BATCH, 1) + bias


# speedup vs baseline: 3.1586x; 1.2134x over previous
import functools

import jax
import jax.numpy as jnp
from jax import lax
from jax.experimental import pallas as pl
from jax.experimental.pallas import tpu as pltpu
from jax.experimental.pallas import tpu_sc as plsc

BATCH = 16384
NUM_FIELDS = 26
NUM_WORKERS = 32
BPW = BATCH // NUM_WORKERS
IDX_PER_W = BPW * NUM_FIELDS

TABLE = 1000000
STAGE = 62464
SUB = STAGE // 4
TAIL = TABLE - 15 * STAGE - 3 * SUB

_mesh = plsc.VectorSubcoreMesh(core_axis_name="c", subcore_axis_name="s")


@functools.partial(
    pl.kernel,
    mesh=_mesh,
    out_type=jax.ShapeDtypeStruct((BATCH,), jnp.float32),
    compiler_params=pltpu.CompilerParams(needs_layout_passes=False),
    scratch_types=[
        pltpu.VMEM((IDX_PER_W,), jnp.int32),
        pltpu.VMEM((IDX_PER_W,), jnp.float32),
        pltpu.VMEM((BPW,), jnp.float32),
        pltpu.SemaphoreType.DMA,
    ],
)
def _emb_sum(xt_hbm, tbl_hbm, out_hbm, idx_v, vals_v, out_v, sem):
    sid = lax.axis_index("s")
    wid = sid * 2 + lax.axis_index("c")
    base = wid * BPW

    for f in range(NUM_FIELDS):
        pltpu.async_copy(xt_hbm.at[f, pl.ds(pl.multiple_of(base, 128), BPW)],
                         idx_v.at[pl.ds(f * BPW, BPW)], sem)
    for f in range(NUM_FIELDS):
        pltpu.make_async_copy(
            xt_hbm.at[f, pl.ds(pl.multiple_of(base, 128), BPW)],
            idx_v.at[pl.ds(f * BPW, BPW)], sem).wait()

    pltpu.async_copy(tbl_hbm.at[0].at[idx_v], vals_v, sem).wait()

    def red(i, c):
        b0 = i * 16
        acc = jnp.zeros((16,), jnp.float32)
        for f in range(NUM_FIELDS):
            acc = acc + vals_v[pl.ds(f * BPW + b0, 16)]
        out_v[pl.ds(b0, 16)] = acc
        return c

    lax.fori_loop(0, BPW // 16, red, 0)

    pltpu.sync_copy(out_v, out_hbm.at[pl.ds(base, BPW)])


def kernel(x, fc_weight, bias):
    xt = x.astype(jnp.int32).T
    tbl = fc_weight.T
    out = _emb_sum(xt, tbl)
    return out.reshape(BATCH, 1) + bias


# speedup vs baseline: 3.2747x; 1.0368x over previous
import functools

import jax
import jax.numpy as jnp
from jax import lax
from jax.experimental import pallas as pl
from jax.experimental.pallas import tpu as pltpu
from jax.experimental.pallas import tpu_sc as plsc

BATCH = 16384
NUM_FIELDS = 26
NUM_WORKERS = 32
BPW = BATCH // NUM_WORKERS
IDX_PER_W = BPW * NUM_FIELDS

TABLE = 1000000
STAGE = 62464
SUB = STAGE // 4
TAIL = TABLE - 15 * STAGE - 3 * SUB

_mesh = plsc.VectorSubcoreMesh(core_axis_name="c", subcore_axis_name="s")


@functools.partial(
    pl.kernel,
    mesh=_mesh,
    out_type=jax.ShapeDtypeStruct((BATCH,), jnp.float32),
    compiler_params=pltpu.CompilerParams(needs_layout_passes=False),
    scratch_types=[
        pltpu.VMEM((IDX_PER_W,), jnp.int32),
        pltpu.VMEM((IDX_PER_W,), jnp.float32),
        pltpu.VMEM((BPW,), jnp.float32),
        pltpu.VMEM((1,), jnp.float32),
        pltpu.SemaphoreType.DMA,
        pltpu.SemaphoreType.DMA,
    ],
)
def _emb_sum(xt_hbm, tbl_hbm, bias_hbm, out_hbm, idx_v, vals_v, out_v, bias_v,
             sem_i, sem_g):
    sid = lax.axis_index("s")
    wid = sid * 2 + lax.axis_index("c")
    base = wid * BPW

    pltpu.async_copy(bias_hbm, bias_v, sem_g)
    for f in range(NUM_FIELDS):
        pltpu.async_copy(xt_hbm.at[f, pl.ds(pl.multiple_of(base, 128), BPW)],
                         idx_v.at[pl.ds(f * BPW, BPW)], sem_i)

    for f in range(NUM_FIELDS):
        pltpu.make_async_copy(
            xt_hbm.at[f, pl.ds(pl.multiple_of(base, 128), BPW)],
            idx_v.at[pl.ds(f * BPW, BPW)], sem_i).wait()
        pltpu.async_copy(tbl_hbm.at[0].at[idx_v.at[pl.ds(f * BPW, BPW)]],
                         vals_v.at[pl.ds(f * BPW, BPW)], sem_g)
    pltpu.make_async_copy(bias_hbm, bias_v, sem_g).wait()
    for f in range(NUM_FIELDS):
        pltpu.make_async_copy(tbl_hbm.at[0].at[idx_v.at[pl.ds(f * BPW, BPW)]],
                              vals_v.at[pl.ds(f * BPW, BPW)], sem_g).wait()

    zeros = jnp.zeros((16,), jnp.int32)
    bias_vec = plsc.load_gather(bias_v, [zeros])

    def red(i, c):
        b0 = i * 16
        acc = bias_vec
        for f in range(NUM_FIELDS):
            acc = acc + vals_v[pl.ds(f * BPW + b0, 16)]
        out_v[pl.ds(b0, 16)] = acc
        return c

    lax.fori_loop(0, BPW // 16, red, 0)

    pltpu.sync_copy(out_v, out_hbm.at[pl.ds(base, BPW)])


def kernel(x, fc_weight, bias):
    xt = x.astype(jnp.int32).T
    tbl = fc_weight.T
    out = _emb_sum(xt, tbl, bias)
    return out.reshape(BATCH, 1)
